# baseline (speedup 1.0000x reference)
import numpy as np
import jax
import jax.numpy as jnp
from jax.experimental import pallas as pl
from jax.experimental.pallas import tpu as pltpu

_T, _V, _RNN, _EMB, _IN, _OUT, _GH, _GW = 16, 32, 128, 64, 9, 5, 19, 5
_C1 = _RNN // 2
_C2 = _RNN // 4
_H1, _W1 = 8, 3
_IJ = _H1 * _W1
_K1 = 15
_H2 = 4
_F2 = _C2 * _H2
_P = _GH * _GW

_ii, _jj = np.meshgrid(np.arange(_H1), np.arange(_W1), indexing="ij")
_kh, _kw = np.meshgrid(np.arange(5), np.arange(3), indexing="ij")
_RR = 2 * _ii.reshape(-1)[:, None] + _kh.reshape(-1)[None, :]
_CC = _jj.reshape(-1)[:, None] + _kw.reshape(-1)[None, :]
_S_np = np.zeros((_P, _IJ * _K1), dtype=np.float32)
for _ij in range(_IJ):
    for _k in range(_K1):
        _S_np[_RR[_ij, _k] * _GW + _CC[_ij, _k], _ij * _K1 + _k] = 1.0

_PERM_np = np.zeros((_F2, _F2), dtype=np.float32)
for _o2 in range(_C2):
    for _i2 in range(_H2):
        _PERM_np[_o2 * _H2 + _i2, _i2 * _C2 + _o2] = 1.0

_BATCH_DIMS = (((2,), (1,)), ((0,), (0,)))


def _body(gc_ref, ga_ref, gb_ref, w1_ref, w2_ref, sp_ref, out_ref, hbuf_ref):
    f32 = jnp.float32
    bf16 = jnp.bfloat16
    x = gc_ref[:, 0:_IN]
    grd = gc_ref[:, _IN:_IN + _P]
    wih = ga_ref[0:4 * _RNN, :]
    whh = ga_ref[4 * _RNN:8 * _RNN, :]
    e2w = ga_ref[8 * _RNN:8 * _RNN + _EMB, :]
    h0 = ga_ref[1088:1088 + _V, :]
    c0 = ga_ref[1120:1120 + _V, :]
    we = ga_ref[1152:1152 + _EMB, 0:_IN]
    wout = ga_ref[1216:1216 + _OUT, :]
    bih = gb_ref[:, 0:4 * _RNN]
    bhh = gb_ref[:, 512:1024]
    be = gb_ref[:, 1024:1088]
    b1r = gb_ref[:, 1088:1152]
    b2 = gb_ref[:, 1152:1184]
    e2b = gb_ref[:, 1184:1248]
    bo = gb_ref[:, 1248:1248 + _OUT]
    s = sp_ref[0:_P, :]
    perm = sp_ref[96:96 + _F2, 0:_F2]
    mflat = (grd != -1.0).astype(bf16)
    mxall = jnp.dot(mflat, s, preferred_element_type=f32)
    mx3 = mxall.astype(bf16).reshape(_T * _V, _IJ, _K1)
    w1rT = w1_ref[:].T.reshape(_RNN, _K1, _C1).astype(bf16).reshape(
        _RNN, _K1 * _C1)
    whhT = whh.T.astype(bf16)
    wbig = jnp.concatenate([w1rT, whhT], axis=1)
    w2band = jnp.transpose(w2_ref[:], (0, 2, 1)).reshape(
        _C2, _K1 * _C1).T.astype(bf16)
    blocks = []
    for i2 in range(_H2):
        top = i2 * _W1 * _C1
        bot = (_H2 - 1) * _W1 * _C1 - top
        parts = []
        if top:
            parts.append(jnp.zeros((top, _C2), bf16))
        parts.append(w2band)
        if bot:
            parts.append(jnp.zeros((bot, _C2), bf16))
        blocks.append(jnp.concatenate(parts, axis=0))
    w2m = jnp.concatenate(blocks, axis=1)
    b2e = jnp.concatenate([b2] * _H2, axis=1)
    e2wPT = jnp.dot(e2w.astype(bf16), perm,
                    preferred_element_type=f32).T.astype(bf16)
    wihT = wih.T
    wihA = wihT[0:_EMB, :].astype(bf16)
    wihB = wihT[_EMB:, :].astype(bf16)
    weT = we.T
    woutT = wout.T
    bg = bih + bhh
    b1 = b1r[:, None, :]
    inp_embs = jax.nn.relu(
        jnp.dot(x, weT, preferred_element_type=f32) + be)
    gx_all = jnp.dot(inp_embs.astype(bf16), wihA, preferred_element_type=f32)
    h = h0.astype(bf16)
    c = c0
    for t in range(_T):
        hp = jnp.dot(h, wbig, preferred_element_type=f32)
        p3 = hp[:, :_C1 * _K1].astype(bf16).reshape(_V, _K1, _C1)
        gh = hp[:, _C1 * _K1:]
        t1 = jax.nn.relu(
            jax.lax.dot_general(mx3[t * _V:(t + 1) * _V], p3, _BATCH_DIMS,
                                preferred_element_type=f32) + b1)
        t1f = t1.astype(bf16).reshape(_V, _IJ * _C1)
        t2 = jax.nn.relu(
            jnp.dot(t1f, w2m, preferred_element_type=f32) + b2e)
        temb = jax.nn.relu(
            jnp.dot(t2.astype(bf16), e2wPT, preferred_element_type=f32)
            + e2b)
        gates = (gx_all[t * _V:(t + 1) * _V, :]
                 + jnp.dot(temb.astype(bf16), wihB, preferred_element_type=f32)
                 + gh + bg)
        i_g = gates[:, 0:_RNN]
        f_g = gates[:, _RNN:2 * _RNN]
        g_g = gates[:, 2 * _RNN:3 * _RNN]
        o_g = gates[:, 3 * _RNN:4 * _RNN]
        c = jax.nn.sigmoid(f_g) * c + jax.nn.sigmoid(i_g) * jnp.tanh(g_g)
        h_new = jax.nn.sigmoid(o_g) * jnp.tanh(c)
        hbuf_ref[t * _V:(t + 1) * _V, :] = h_new
        h = h_new.astype(bf16)
    out_all = (jnp.dot(hbuf_ref[:], woutT, preferred_element_type=f32)
               + bo)
    out_ref[:] = out_all.reshape(_T, _V, _OUT)


_SP_np = np.zeros((224, _IJ * _K1), dtype=np.float32)
_SP_np[0:_P, :] = _S_np
_SP_np[96:96 + _F2, 0:_F2] = _PERM_np


def kernel(x_seq, grids, hidden_states, cell_states, W_embed, b_embed,
           conv1_w, conv1_b, conv2_w, conv2_b, embed2_w, embed2_b,
           W_ih, W_hh, b_ih, b_hh, W_out, b_out):
    f32 = jnp.float32
    gc = jnp.concatenate(
        [x_seq.reshape(_T * _V, _IN), grids.reshape(_T * _V, _P)], axis=1)
    ga = jnp.concatenate(
        [W_ih, W_hh, embed2_w, hidden_states, cell_states,
         jnp.pad(W_embed, ((0, 0), (0, _F2 - _IN))), W_out], axis=0)
    gb = jnp.concatenate(
        [b_ih, b_hh, b_embed, conv1_b, conv2_b, embed2_b, b_out])[None, :]
    out = pl.pallas_call(
        _body,
        out_shape=jax.ShapeDtypeStruct((_T, _V, _OUT), f32),
        scratch_shapes=[pltpu.VMEM((_T * _V, _RNN), f32)],
    )(
        gc, ga, gb,
        conv1_w.reshape(_C1, _RNN * _K1), conv2_w.reshape(_C2, _C1, _K1),
        jnp.asarray(_SP_np, jnp.bfloat16),
    )
    return out


# speedup vs baseline: 1.0929x; 1.0929x over previous
import numpy as np
import jax
import jax.numpy as jnp
from jax.experimental import pallas as pl
from jax.experimental.pallas import tpu as pltpu

_T, _V, _RNN, _EMB, _IN, _OUT, _GH, _GW = 16, 32, 128, 64, 9, 5, 19, 5
_C1 = _RNN // 2
_C2 = _RNN // 4
_H1, _W1 = 8, 3
_IJ = _H1 * _W1
_K1 = 15
_H2 = 4
_F2 = _C2 * _H2
_P = _GH * _GW

_ii, _jj = np.meshgrid(np.arange(_H1), np.arange(_W1), indexing="ij")
_kh, _kw = np.meshgrid(np.arange(5), np.arange(3), indexing="ij")
_RR = 2 * _ii.reshape(-1)[:, None] + _kh.reshape(-1)[None, :]
_CC = _jj.reshape(-1)[:, None] + _kw.reshape(-1)[None, :]
_S_np = np.zeros((_P, _IJ * _K1), dtype=np.float32)
for _ij in range(_IJ):
    for _k in range(_K1):
        _S_np[_RR[_ij, _k] * _GW + _CC[_ij, _k], _ij * _K1 + _k] = 1.0

_PERM_np = np.zeros((_F2, _F2), dtype=np.float32)
for _o2 in range(_C2):
    for _i2 in range(_H2):
        _PERM_np[_o2 * _H2 + _i2, _i2 * _C2 + _o2] = 1.0

_BATCH_DIMS = (((2,), (1,)), ((0,), (0,)))


def _body(x_ref, g_ref, h0_ref, c0_ref, we_ref, be_ref, w1_ref, b1_ref,
          w2_ref, b2_ref, e2w_ref, e2b_ref, wih_ref, whh_ref, bih_ref,
          bhh_ref, wout_ref, bo_ref, s_ref, perm_ref, out_ref, hbuf_ref,
          mxs_ref, gxs_ref):
    f32 = jnp.float32
    bf16 = jnp.bfloat16
    mflat = (g_ref[:] != -1.0).astype(bf16)
    mxall = jnp.dot(mflat, s_ref[:], preferred_element_type=f32)
    mx3 = mxall.astype(bf16).reshape(_T * _V, _IJ, _K1)
    w1rT = w1_ref[:].T.reshape(_RNN, _K1, _C1).astype(bf16).reshape(
        _RNN, _K1 * _C1)
    whhT = whh_ref[:].T.astype(bf16)
    wbig = jnp.concatenate([w1rT, whhT], axis=1)
    w2band = jnp.transpose(w2_ref[:], (0, 2, 1)).reshape(
        _C2, _K1 * _C1).T.astype(bf16)
    blocks = []
    for i2 in range(_H2):
        top = i2 * _W1 * _C1
        bot = (_H2 - 1) * _W1 * _C1 - top
        parts = []
        if top:
            parts.append(jnp.zeros((top, _C2), bf16))
        parts.append(w2band)
        if bot:
            parts.append(jnp.zeros((bot, _C2), bf16))
        blocks.append(jnp.concatenate(parts, axis=0))
    w2m = jnp.concatenate(blocks, axis=1)
    b2e = jnp.concatenate([b2_ref[:]] * _H2, axis=1)
    e2wPT = jnp.dot(e2w_ref[:].astype(bf16), perm_ref[:],
                    preferred_element_type=f32).T.astype(bf16)
    wihT = wih_ref[:].T
    wihA = wihT[0:_EMB, :].astype(bf16)
    wihB = wihT[_EMB:, :].astype(bf16)
    weT = we_ref[:].T
    woutT = wout_ref[:].T
    bg = bih_ref[:] + bhh_ref[:]
    b1 = b1_ref[:][:, None, :]
    inp_embs = jax.nn.relu(
        jnp.dot(x_ref[:], weT, preferred_element_type=f32) + be_ref[:])
    gx_all = jnp.dot(inp_embs.astype(bf16), wihA, preferred_element_type=f32)
    mxs_ref[:] = mx3
    gxs_ref[:] = gx_all

    def _step(t, carry):
        h, c = carry
        hp = jnp.dot(h, wbig, preferred_element_type=f32)
        p3 = hp[:, :_C1 * _K1].astype(bf16).reshape(_V, _K1, _C1)
        gh = hp[:, _C1 * _K1:]
        t1 = jax.nn.relu(
            jax.lax.dot_general(mxs_ref[pl.ds(t * _V, _V)], p3, _BATCH_DIMS,
                                preferred_element_type=f32) + b1)
        t1f = t1.astype(bf16).reshape(_V, _IJ * _C1)
        t2 = jax.nn.relu(
            jnp.dot(t1f, w2m, preferred_element_type=f32) + b2e)
        temb = jax.nn.relu(
            jnp.dot(t2.astype(bf16), e2wPT, preferred_element_type=f32)
            + e2b_ref[:])
        gates = (gxs_ref[pl.ds(t * _V, _V)]
                 + jnp.dot(temb.astype(bf16), wihB, preferred_element_type=f32)
                 + gh + bg)
        i_g = gates[:, 0:_RNN]
        f_g = gates[:, _RNN:2 * _RNN]
        g_g = gates[:, 2 * _RNN:3 * _RNN]
        o_g = gates[:, 3 * _RNN:4 * _RNN]
        c = jax.nn.sigmoid(f_g) * c + jax.nn.sigmoid(i_g) * jnp.tanh(g_g)
        h_new = jax.nn.sigmoid(o_g) * jnp.tanh(c)
        hbuf_ref[pl.ds(t * _V, _V), :] = h_new
        return h_new.astype(bf16), c

    h, c = jax.lax.fori_loop(0, _T, _step, (h0_ref[:].astype(bf16), c0_ref[:]))
    out_all = (jnp.dot(hbuf_ref[:], woutT, preferred_element_type=f32)
               + bo_ref[:])
    out_ref[:] = out_all.reshape(_T, _V, _OUT)


def kernel(x_seq, grids, hidden_states, cell_states, W_embed, b_embed,
           conv1_w, conv1_b, conv2_w, conv2_b, embed2_w, embed2_b,
           W_ih, W_hh, b_ih, b_hh, W_out, b_out):
    f32 = jnp.float32
    out = pl.pallas_call(
        _body,
        out_shape=jax.ShapeDtypeStruct((_T, _V, _OUT), f32),
        scratch_shapes=[pltpu.VMEM((_T * _V, _RNN), f32),
                        pltpu.VMEM((_T * _V, _IJ, _K1), jnp.bfloat16),
                        pltpu.VMEM((_T * _V, 4 * _RNN), f32)],
    )(
        x_seq.reshape(_T * _V, _IN), grids.reshape(_T * _V, _P),
        hidden_states, cell_states,
        W_embed, b_embed[None, :],
        conv1_w.reshape(_C1, _RNN * _K1), conv1_b[None, :],
        conv2_w.reshape(_C2, _C1, _K1), conv2_b[None, :],
        embed2_w, embed2_b[None, :],
        W_ih, W_hh, b_ih[None, :], b_hh[None, :],
        W_out, b_out[None, :],
        jnp.asarray(_S_np, jnp.bfloat16), jnp.asarray(_PERM_np, jnp.bfloat16),
    )
    return out


# speedup vs baseline: 1.1549x; 1.0567x over previous
import numpy as np
import jax
import jax.numpy as jnp
from jax.experimental import pallas as pl
from jax.experimental.pallas import tpu as pltpu

_T, _V, _RNN, _EMB, _IN, _OUT, _GH, _GW = 16, 32, 128, 64, 9, 5, 19, 5
_C1 = _RNN // 2
_C2 = _RNN // 4
_H1, _W1 = 8, 3
_IJ = _H1 * _W1
_K1 = 15
_H2 = 4
_F2 = _C2 * _H2
_P = _GH * _GW

_ii, _jj = np.meshgrid(np.arange(_H1), np.arange(_W1), indexing="ij")
_kh, _kw = np.meshgrid(np.arange(5), np.arange(3), indexing="ij")
_RR = 2 * _ii.reshape(-1)[:, None] + _kh.reshape(-1)[None, :]
_CC = _jj.reshape(-1)[:, None] + _kw.reshape(-1)[None, :]
_S_np = np.zeros((_P, _IJ * _K1), dtype=np.float32)
for _ij in range(_IJ):
    for _k in range(_K1):
        _S_np[_RR[_ij, _k] * _GW + _CC[_ij, _k], _ij * _K1 + _k] = 1.0

_PERM_np = np.zeros((_F2, _F2), dtype=np.float32)
for _o2 in range(_C2):
    for _i2 in range(_H2):
        _PERM_np[_o2 * _H2 + _i2, _i2 * _C2 + _o2] = 1.0

_BATCH_DIMS = (((2,), (1,)), ((0,), (0,)))


def _body(x_ref, g_ref, h0_ref, c0_ref, we_ref, be_ref, w1_ref, b1_ref,
          w2_ref, b2_ref, e2w_ref, e2b_ref, wih_ref, whh_ref, bih_ref,
          bhh_ref, wout_ref, bo_ref, s_ref, perm_ref, out_ref, hbuf_ref):
    f32 = jnp.float32
    bf16 = jnp.bfloat16
    mflat = (g_ref[:] != -1.0).astype(bf16)
    mxall = jnp.dot(mflat, s_ref[:], preferred_element_type=f32)
    mx3 = mxall.astype(bf16).reshape(_T * _V, _IJ, _K1)
    w1rT = w1_ref[:].T.reshape(_RNN, _K1, _C1).astype(bf16).reshape(
        _RNN, _K1 * _C1)
    whhT = whh_ref[:].T.astype(bf16)
    wbig = jnp.concatenate([w1rT, whhT], axis=1)
    w2band = jnp.transpose(w2_ref[:], (0, 2, 1)).reshape(
        _C2, _K1 * _C1).T.astype(bf16)
    blocks = []
    for i2 in range(_H2):
        top = i2 * _W1 * _C1
        bot = (_H2 - 1) * _W1 * _C1 - top
        parts = []
        if top:
            parts.append(jnp.zeros((top, _C2), bf16))
        parts.append(w2band)
        if bot:
            parts.append(jnp.zeros((bot, _C2), bf16))
        blocks.append(jnp.concatenate(parts, axis=0))
    w2m = jnp.concatenate(blocks, axis=1)
    b2e = jnp.concatenate([b2_ref[:]] * _H2, axis=1)
    e2wPT = jnp.dot(e2w_ref[:].astype(bf16), perm_ref[:],
                    preferred_element_type=f32).T.astype(bf16)
    wihT = wih_ref[:].T
    wihA = wihT[0:_EMB, :].astype(bf16)
    wihB = wihT[_EMB:, :].astype(bf16)
    weT = we_ref[:].T
    woutT = wout_ref[:].T
    bg = bih_ref[:] + bhh_ref[:]
    b1 = b1_ref[:][:, None, :]
    inp_embs = jax.nn.relu(
        jnp.dot(x_ref[:], weT, preferred_element_type=f32) + be_ref[:])
    gx_all = jnp.dot(inp_embs.astype(bf16), wihA, preferred_element_type=f32)
    h = h0_ref[:].astype(bf16)
    c = c0_ref[:]
    for t in range(_T):
        hp = jnp.dot(h, wbig, preferred_element_type=f32)
        p3 = hp[:, :_C1 * _K1].astype(bf16).reshape(_V, _K1, _C1)
        gh = hp[:, _C1 * _K1:]
        t1 = jax.nn.relu(
            jax.lax.dot_general(mx3[t * _V:(t + 1) * _V], p3, _BATCH_DIMS,
                                preferred_element_type=f32) + b1)
        t1f = t1.astype(bf16).reshape(_V, _IJ * _C1)
        t2 = jax.nn.relu(
            jnp.dot(t1f, w2m, preferred_element_type=f32) + b2e)
        temb = jax.nn.relu(
            jnp.dot(t2.astype(bf16), e2wPT, preferred_element_type=f32)
            + e2b_ref[:])
        gates = (gx_all[t * _V:(t + 1) * _V, :]
                 + jnp.dot(temb.astype(bf16), wihB, preferred_element_type=f32)
                 + gh + bg)
        i_g = gates[:, 0:_RNN]
        f_g = gates[:, _RNN:2 * _RNN]
        g_g = gates[:, 2 * _RNN:3 * _RNN]
        o_g = gates[:, 3 * _RNN:4 * _RNN]
        c = jax.nn.sigmoid(f_g) * c + jax.nn.sigmoid(i_g) * jnp.tanh(g_g)
        h_new = jax.nn.sigmoid(o_g) * jnp.tanh(c)
        hbuf_ref[t * _V:(t + 1) * _V, :] = h_new
        h = h_new.astype(bf16)
    out_all = (jnp.dot(hbuf_ref[:], woutT, preferred_element_type=f32)
               + bo_ref[:])
    out_ref[:] = out_all.reshape(_T, _V, _OUT)


def kernel(x_seq, grids, hidden_states, cell_states, W_embed, b_embed,
           conv1_w, conv1_b, conv2_w, conv2_b, embed2_w, embed2_b,
           W_ih, W_hh, b_ih, b_hh, W_out, b_out):
    f32 = jnp.float32
    out = pl.pallas_call(
        _body,
        out_shape=jax.ShapeDtypeStruct((_T, _V, _OUT), f32),
        scratch_shapes=[pltpu.VMEM((_T * _V, _RNN), f32)],
    )(
        x_seq.reshape(_T * _V, _IN), grids.reshape(_T * _V, _P),
        hidden_states, cell_states,
        W_embed, b_embed[None, :],
        conv1_w.reshape(_C1, _RNN * _K1), conv1_b[None, :],
        conv2_w.reshape(_C2, _C1, _K1), conv2_b[None, :],
        embed2_w, embed2_b[None, :],
        W_ih, W_hh, b_ih[None, :], b_hh[None, :],
        W_out, b_out[None, :],
        jnp.asarray(_S_np, jnp.bfloat16), jnp.asarray(_PERM_np, jnp.bfloat16),
    )
    return out
